# Initial kernel scaffold; baseline (speedup 1.0000x reference)
import jax
import jax.numpy as jnp
from jax.experimental import pallas as pl


def kernel(batch_x, Wi_m, bi_m, Wr_m, br_m, Wo_m, bo_m, Wi_s, bi_s, Wr_s, br_s, Wo_s, bo_s, Wf1, bf1, Wf2, bf2, weight):
    raise NotImplementedError("write your pallas kernel here")


# speedup vs baseline: 17.0175x; 17.0175x over previous
import functools

import jax
import jax.numpy as jnp
import numpy as np
from jax import lax
from jax.experimental import pallas as pl

SEQ_LEN = 720
ENC_IN = 321
PERIOD = 12
NGROUP = SEQ_LEN // PERIOD
NGROUP_PAD = 64
TOPK = 30
NBIN = SEQ_LEN // 2 + 1
NBIN_PAD = 368
EPS = 1e-8
_INF_BITS = 0x7F800000


_HI = jax.lax.Precision.HIGHEST


def _dft_body(x_ref, a_ref, e_ref, ccos_ref, csin_ref, tcos_ref, tsin_ref,
              out_ref):
    x = x_ref[0]
    a = a_ref[...]
    mean = jnp.dot(a, x, preferred_element_type=jnp.float32, precision=_HI)
    msq = jnp.dot(a, x * x, preferred_element_type=jnp.float32, precision=_HI)
    var = (msq - mean * mean) * (PERIOD / (PERIOD - 1))
    rinv = 1.0 / (jnp.sqrt(jnp.maximum(var, 0.0)) + EPS)
    e = e_ref[...]
    mb = jnp.dot(e, mean, preferred_element_type=jnp.float32, precision=_HI)
    rb = jnp.dot(e, rinv, preferred_element_type=jnp.float32, precision=_HI)
    ni = (x - mb) * rb

    re = jnp.dot(ccos_ref[...], ni, preferred_element_type=jnp.float32, precision=_HI)
    s = jnp.dot(csin_ref[...], ni, preferred_element_type=jnp.float32, precision=_HI)
    mag2 = re * re + s * s
    bits = lax.bitcast_convert_type(mag2, jnp.int32)

    j = lax.broadcasted_iota(jnp.int32, (NBIN_PAD, 1), 0)
    mu = jnp.where((j == 0) | (j == NBIN - 1), 1.0,
                   jnp.where(j < NBIN, 2.0, 0.0))

    def body(_, carry):
        lo, hi = carry
        mid = lo + lax.shift_right_logical(hi - lo, 1)
        cnt = jnp.sum(jnp.where(bits >= mid, mu, 0.0), axis=0, keepdims=True)
        ge = cnt >= float(TOPK)
        return jnp.where(ge, mid, lo), jnp.where(ge, hi, mid)

    lo0 = jnp.zeros((1, ENC_IN), jnp.int32)
    hi0 = jnp.full((1, ENC_IN), _INF_BITS, jnp.int32)
    lo, _ = lax.fori_loop(0, 31, body, (lo0, hi0))

    gt = bits > lo
    cnt_gt = jnp.sum(jnp.where(gt, mu, 0.0), axis=0, keepdims=True)
    rem = float(TOPK) - cnt_gt
    w = jnp.where(gt, mu, 0.0) + jnp.where(bits == lo, rem, 0.0)

    wre = (w * re).astype(jnp.bfloat16)
    ws = (w * s).astype(jnp.bfloat16)
    xf = (jnp.dot(tcos_ref[...], wre, preferred_element_type=jnp.float32) +
          jnp.dot(tsin_ref[...], ws, preferred_element_type=jnp.float32))
    out_ref[0] = ni - xf


@functools.lru_cache(maxsize=1)
def _consts():
    n = np.arange(SEQ_LEN)
    j = np.arange(NBIN_PAD)
    ang = 2.0 * np.pi * np.outer(j, n) / SEQ_LEN
    ccos = np.cos(ang)
    csin = np.sin(ang)
    ccos[NBIN:] = 0.0
    csin[NBIN:] = 0.0
    tcos = (ccos.T / SEQ_LEN).astype(np.float32)
    tsin = (csin.T / SEQ_LEN).astype(np.float32)
    a = np.zeros((NGROUP_PAD, SEQ_LEN), np.float32)
    for g in range(NGROUP):
        a[g, g * PERIOD:(g + 1) * PERIOD] = 1.0 / PERIOD
    e = np.zeros((SEQ_LEN, NGROUP_PAD), np.float32)
    e[n, n // PERIOD] = 1.0
    return (jnp.asarray(a), jnp.asarray(e),
            jnp.asarray(ccos.astype(np.float32)),
            jnp.asarray(csin.astype(np.float32)),
            jnp.asarray(tcos, dtype=jnp.bfloat16),
            jnp.asarray(tsin, dtype=jnp.bfloat16))


def kernel(batch_x, Wi_m, bi_m, Wr_m, br_m, Wo_m, bo_m, Wi_s, bi_s, Wr_s,
           br_s, Wo_s, bo_s, Wf1, bf1, Wf2, bf2, weight):
    bs = batch_x.shape[0]
    a, e, ccos, csin, tcos, tsin = _consts()
    full = lambda shape: pl.BlockSpec(shape, lambda b: (0,) * len(shape))
    out = pl.pallas_call(
        _dft_body,
        grid=(bs,),
        in_specs=[
            pl.BlockSpec((1, SEQ_LEN, ENC_IN), lambda b: (b, 0, 0)),
            full((NGROUP_PAD, SEQ_LEN)),
            full((SEQ_LEN, NGROUP_PAD)),
            full((NBIN_PAD, SEQ_LEN)),
            full((NBIN_PAD, SEQ_LEN)),
            full((SEQ_LEN, NBIN_PAD)),
            full((SEQ_LEN, NBIN_PAD)),
        ],
        out_specs=pl.BlockSpec((1, SEQ_LEN, ENC_IN), lambda b: (b, 0, 0)),
        out_shape=jax.ShapeDtypeStruct((bs, SEQ_LEN, ENC_IN), jnp.float32),
    )(batch_x, a, e, ccos, csin, tcos, tsin)
    return out


# speedup vs baseline: 23.8511x; 1.4016x over previous
import functools

import jax
import jax.numpy as jnp
import numpy as np
from jax import lax
from jax.experimental import pallas as pl

SEQ_LEN = 720
ENC_IN = 321
PERIOD = 12
NGROUP = SEQ_LEN // PERIOD
NGROUP_PAD = 64
TOPK = 30
NBIN = SEQ_LEN // 2 + 1
NBIN_PAD = 368
EPS = 1e-8
_INF_BITS = 0x7F800000

_BF = jnp.bfloat16
_F32 = jnp.float32


def _split(v):
    hi = v.astype(_BF)
    lo = (v - hi.astype(_F32)).astype(_BF)
    return hi, lo


def _mm(a, b):
    return jnp.dot(a, b, preferred_element_type=_F32)


def _mm_x2(ahi, alo, bhi, blo):
    return _mm(ahi, bhi) + (_mm(ahi, blo) + _mm(alo, bhi))


def _dft_body(x_ref, a_ref, e_ref, chi_ref, clo_ref, shi_ref, slo_ref,
              tcos_ref, tsin_ref, out_ref):
    x = x_ref[0]
    xhi, xlo = _split(x)
    x2 = x * x
    x2hi, x2lo = _split(x2)
    a = a_ref[...]
    mean = (_mm(a, xhi) + _mm(a, xlo)) * (1.0 / PERIOD)
    msq = (_mm(a, x2hi) + _mm(a, x2lo)) * (1.0 / PERIOD)
    var = (msq - mean * mean) * (PERIOD / (PERIOD - 1))
    rinv = 1.0 / (jnp.sqrt(jnp.maximum(var, 0.0)) + EPS)
    e = e_ref[...]
    mhi, mlo = _split(mean)
    rhi, rlo = _split(rinv)
    mb = _mm(e, mhi) + _mm(e, mlo)
    rb = _mm(e, rhi) + _mm(e, rlo)
    ni = (x - mb) * rb

    nhi, nlo = _split(ni)
    re = _mm_x2(chi_ref[...], clo_ref[...], nhi, nlo)
    s = _mm_x2(shi_ref[...], slo_ref[...], nhi, nlo)
    mag2 = re * re + s * s
    bits = lax.bitcast_convert_type(mag2, jnp.int32)

    j = lax.broadcasted_iota(jnp.int32, (NBIN_PAD, 1), 0)
    mu = jnp.where((j == 0) | (j == NBIN - 1), 1.0,
                   jnp.where(j < NBIN, 2.0, 0.0))

    def body(_, carry):
        lo, hi = carry
        mid = lo + lax.shift_right_logical(hi - lo, 1)
        cnt = jnp.sum(jnp.where(bits >= mid, mu, 0.0), axis=0, keepdims=True)
        ge = cnt >= float(TOPK)
        return jnp.where(ge, mid, lo), jnp.where(ge, hi, mid)

    lo0 = jnp.zeros((1, ENC_IN), jnp.int32)
    hi0 = jnp.full((1, ENC_IN), _INF_BITS, jnp.int32)
    lo, _ = lax.fori_loop(0, 31, body, (lo0, hi0))

    gt = bits > lo
    cnt_gt = jnp.sum(jnp.where(gt, mu, 0.0), axis=0, keepdims=True)
    rem = float(TOPK) - cnt_gt
    w = jnp.where(gt, mu, 0.0) + jnp.where(bits == lo, rem, 0.0)

    wre = (w * re).astype(_BF)
    ws = (w * s).astype(_BF)
    xf = _mm(tcos_ref[...], wre) + _mm(tsin_ref[...], ws)
    out_ref[0] = ni - xf


@functools.lru_cache(maxsize=1)
def _consts():
    n = np.arange(SEQ_LEN)
    j = np.arange(NBIN_PAD)
    ang = 2.0 * np.pi * np.outer(j, n) / SEQ_LEN
    ccos = np.cos(ang)
    csin = np.sin(ang)
    ccos[NBIN:] = 0.0
    csin[NBIN:] = 0.0
    tcos = (ccos.T / SEQ_LEN).astype(np.float32)
    tsin = (csin.T / SEQ_LEN).astype(np.float32)

    def split(m):
        m32 = m.astype(np.float32)
        hi = m32.astype(jnp.bfloat16)
        lo = (m32 - np.asarray(hi, np.float32)).astype(jnp.bfloat16)
        return jnp.asarray(hi), jnp.asarray(lo)

    chi, clo = split(ccos)
    shi, slo = split(csin)
    a = np.zeros((NGROUP_PAD, SEQ_LEN), np.float32)
    for g in range(NGROUP):
        a[g, g * PERIOD:(g + 1) * PERIOD] = 1.0
    e = np.zeros((SEQ_LEN, NGROUP_PAD), np.float32)
    e[n, n // PERIOD] = 1.0
    return (jnp.asarray(a, dtype=jnp.bfloat16),
            jnp.asarray(e, dtype=jnp.bfloat16),
            chi, clo, shi, slo,
            jnp.asarray(tcos, dtype=jnp.bfloat16),
            jnp.asarray(tsin, dtype=jnp.bfloat16))


def kernel(batch_x, Wi_m, bi_m, Wr_m, br_m, Wo_m, bo_m, Wi_s, bi_s, Wr_s,
           br_s, Wo_s, bo_s, Wf1, bf1, Wf2, bf2, weight):
    bs = batch_x.shape[0]
    a, e, chi, clo, shi, slo, tcos, tsin = _consts()
    full = lambda shape: pl.BlockSpec(shape, lambda b: (0,) * len(shape))
    out = pl.pallas_call(
        _dft_body,
        grid=(bs,),
        in_specs=[
            pl.BlockSpec((1, SEQ_LEN, ENC_IN), lambda b: (b, 0, 0)),
            full((NGROUP_PAD, SEQ_LEN)),
            full((SEQ_LEN, NGROUP_PAD)),
            full((NBIN_PAD, SEQ_LEN)),
            full((NBIN_PAD, SEQ_LEN)),
            full((NBIN_PAD, SEQ_LEN)),
            full((NBIN_PAD, SEQ_LEN)),
            full((SEQ_LEN, NBIN_PAD)),
            full((SEQ_LEN, NBIN_PAD)),
        ],
        out_specs=pl.BlockSpec((1, SEQ_LEN, ENC_IN), lambda b: (b, 0, 0)),
        out_shape=jax.ShapeDtypeStruct((bs, SEQ_LEN, ENC_IN), jnp.float32),
    )(batch_x, a, e, chi, clo, shi, slo, tcos, tsin)
    return out
